# baseline (speedup 1.0000x reference)
import jax
import jax.numpy as jnp
from jax import lax
from jax.experimental import pallas as pl
from jax.experimental.pallas import tpu as pltpu
from jax.experimental.pallas import tpu_sc as plsc

_VOCAB = 100000
_B = 4096
_M = 100
_TAB_PAD = 102400

_info = plsc.get_sparse_core_info()
_NC, _NS, _L = _info.num_cores, _info.num_subcores, _info.num_lanes
_NW = _NC * _NS
_ROWS = _B // _NW
_GROUPS = _ROWS // _L


def _sc_body(idx_hbm, tab_hbm, out_hbm, idx_v, acc_v, tab_sh, sem_t, sem_i):
    sid = lax.axis_index("s")
    wid = sid * _NC + lax.axis_index("c")
    base = wid * _ROWS

    cp_idx = pltpu.async_copy(idx_hbm.at[:, pl.ds(base, _ROWS)], idx_v, sem_i)

    for g in range(_GROUPS):
        acc_v[pl.ds(g * _L, _L)] = jnp.zeros((_L,), jnp.float32)

    @pl.when(sid == 0)
    def _():
        pltpu.sync_copy(tab_hbm, tab_sh)
        pltpu.sync_copy(acc_v.at[pl.ds(0, 8)], tab_sh.at[pl.ds(_VOCAB, 8)])

    plsc.subcore_barrier()
    cp_idx.wait()

    cps = [
        pltpu.async_copy(tab_sh.at[idx_v.at[m]], acc_v, sem_t, add=True)
        for m in range(_M)
    ]
    for cp in cps:
        cp.wait()

    pltpu.sync_copy(acc_v, out_hbm.at[pl.ds(base, _ROWS)])


@jax.jit
def _sc_call(idx_t, tab):
    mesh = plsc.VectorSubcoreMesh(core_axis_name="c", subcore_axis_name="s")
    return pl.kernel(
        _sc_body,
        mesh=mesh,
        out_type=jax.ShapeDtypeStruct((_B,), jnp.float32),
        compiler_params=pltpu.CompilerParams(needs_layout_passes=False),
        scratch_types=[
            pltpu.VMEM((_M, _ROWS), jnp.int32),
            pltpu.VMEM((_ROWS,), jnp.float32),
            pltpu.VMEM_SHARED((_TAB_PAD,), jnp.float32),
            pltpu.SemaphoreType.DMA,
            pltpu.SemaphoreType.DMA,
        ],
    )(idx_t, tab)


def kernel(inputs, table):
    tab = jnp.pad(table, ((0, _TAB_PAD - (_VOCAB + 1)), (0, 0)))
    return _sc_call(inputs.T, tab.reshape(-1))[:, None]


# speedup vs baseline: 1.0338x; 1.0338x over previous
import jax
import jax.numpy as jnp
from jax import lax
from jax.experimental import pallas as pl
from jax.experimental.pallas import tpu as pltpu
from jax.experimental.pallas import tpu_sc as plsc

_VOCAB = 100000
_B = 4096
_M = 100
_TAB_PAD = 102400
_SPLIT = 94208
_HALF = _SPLIT // 2

_info = plsc.get_sparse_core_info()
_NC, _NS, _L = _info.num_cores, _info.num_subcores, _info.num_lanes
_NW = _NC * _NS
_ROWS = _B // _NW
_GROUPS = _ROWS // _L


def _sc_body(idx_hbm, tab_hbm, out_hbm, idx_v, tab_v, tab_sh, out_v,
             sem_t, sem_i, sem_d):
    sid = lax.axis_index("s")
    wid = sid * _NC + lax.axis_index("c")
    base = wid * _ROWS

    cp_idx = pltpu.async_copy(idx_hbm.at[:, pl.ds(base, _ROWS)], idx_v, sem_i)
    cp_dir = pltpu.async_copy(
        tab_hbm.at[pl.ds(_SPLIT, _TAB_PAD - _SPLIT)],
        tab_v.at[pl.ds(_SPLIT, _TAB_PAD - _SPLIT)], sem_d)

    @pl.when(sid == 0)
    def _():
        pltpu.sync_copy(tab_hbm.at[pl.ds(0, _HALF)], tab_sh.at[pl.ds(0, _HALF)])

    plsc.subcore_barrier()
    cp_t0 = pltpu.async_copy(
        tab_sh.at[pl.ds(0, _HALF)], tab_v.at[pl.ds(0, _HALF)], sem_t)

    @pl.when(sid == 0)
    def _():
        pltpu.sync_copy(tab_hbm.at[pl.ds(_HALF, _SPLIT - _HALF)],
                        tab_sh.at[pl.ds(_HALF, _SPLIT - _HALF)])

    plsc.subcore_barrier()
    cp_t1 = pltpu.async_copy(
        tab_sh.at[pl.ds(_HALF, _SPLIT - _HALF)],
        tab_v.at[pl.ds(_HALF, _SPLIT - _HALF)], sem_t)
    cp_t0.wait()
    cp_t1.wait()
    cp_dir.wait()
    tab_v[pl.ds(_VOCAB, _L)] = jnp.zeros((_L,), jnp.float32)
    cp_idx.wait()

    zeros = tuple(jnp.zeros((_L,), jnp.float32) for _ in range(_GROUPS))

    @plsc.parallel_loop(0, _M, unroll=4, carry=zeros)
    def accs(m, accs_in):
        out = []
        for r in range(_GROUPS):
            ids = idx_v[m, pl.ds(r * _L, _L)]
            vals = plsc.load_gather(tab_v, [ids])
            out.append(accs_in[r] + vals)
        return tuple(out)

    for r in range(_GROUPS):
        out_v[pl.ds(r * _L, _L)] = accs[r]

    pltpu.sync_copy(out_v, out_hbm.at[pl.ds(base, _ROWS)])


@jax.jit
def _sc_call(idx_t, tab):
    mesh = plsc.VectorSubcoreMesh(core_axis_name="c", subcore_axis_name="s")
    return pl.kernel(
        _sc_body,
        mesh=mesh,
        out_type=jax.ShapeDtypeStruct((_B,), jnp.float32),
        compiler_params=pltpu.CompilerParams(needs_layout_passes=False),
        scratch_types=[
            pltpu.VMEM((_M, _ROWS), jnp.int32),
            pltpu.VMEM((_TAB_PAD,), jnp.float32),
            pltpu.VMEM_SHARED((_SPLIT,), jnp.float32),
            pltpu.VMEM((_ROWS,), jnp.float32),
            pltpu.SemaphoreType.DMA,
            pltpu.SemaphoreType.DMA,
            pltpu.SemaphoreType.DMA,
        ],
    )(idx_t, tab)


def kernel(inputs, table):
    tab = jnp.pad(table, ((0, _TAB_PAD - (_VOCAB + 1)), (0, 0)))
    return _sc_call(inputs.T, tab.reshape(-1))[:, None]


# speedup vs baseline: 1.0539x; 1.0194x over previous
import jax
import jax.numpy as jnp
from jax import lax
from jax.experimental import pallas as pl
from jax.experimental.pallas import tpu as pltpu
from jax.experimental.pallas import tpu_sc as plsc

_VOCAB = 100000
_B = 4096
_M = 100
_TAB_PAD = 102400

_info = plsc.get_sparse_core_info()
_NC, _NS, _L = _info.num_cores, _info.num_subcores, _info.num_lanes
_NW = _NC * _NS
_ROWS = _B // _NW
_GROUPS = _ROWS // _L


def _sc_body(idx_hbm, tab_hbm, out_hbm, idx_v, tab_v, tab_sh, out_v,
             sem_t, sem_i):
    sid = lax.axis_index("s")
    wid = sid * _NC + lax.axis_index("c")
    base = wid * _ROWS

    cp_idx = pltpu.async_copy(idx_hbm.at[:, pl.ds(base, _ROWS)], idx_v, sem_i)

    @pl.when(sid == 0)
    def _():
        pltpu.sync_copy(tab_hbm, tab_sh)

    plsc.subcore_barrier()
    cp_tab = pltpu.async_copy(tab_sh, tab_v, sem_t)
    cp_tab.wait()
    tab_v[pl.ds(_VOCAB, _L)] = jnp.zeros((_L,), jnp.float32)
    cp_idx.wait()

    zeros = tuple(jnp.zeros((_L,), jnp.float32) for _ in range(_GROUPS))

    @plsc.parallel_loop(0, _M, unroll=4, carry=zeros)
    def accs(m, accs_in):
        out = []
        for r in range(_GROUPS):
            ids = idx_v[m, pl.ds(r * _L, _L)]
            vals = plsc.load_gather(tab_v, [ids])
            out.append(accs_in[r] + vals)
        return tuple(out)

    for r in range(_GROUPS):
        out_v[pl.ds(r * _L, _L)] = accs[r]

    pltpu.sync_copy(out_v, out_hbm.at[pl.ds(base, _ROWS)])


@jax.jit
def _sc_call(idx_t, tab):
    mesh = plsc.VectorSubcoreMesh(core_axis_name="c", subcore_axis_name="s")
    return pl.kernel(
        _sc_body,
        mesh=mesh,
        out_type=jax.ShapeDtypeStruct((_B,), jnp.float32),
        compiler_params=pltpu.CompilerParams(needs_layout_passes=False),
        scratch_types=[
            pltpu.VMEM((_M, _ROWS), jnp.int32),
            pltpu.VMEM((_TAB_PAD,), jnp.float32),
            pltpu.VMEM_SHARED((_TAB_PAD,), jnp.float32),
            pltpu.VMEM((_ROWS,), jnp.float32),
            pltpu.SemaphoreType.DMA,
            pltpu.SemaphoreType.DMA,
        ],
    )(idx_t, tab)


def kernel(inputs, table):
    tab = jnp.concatenate(
        [table, jnp.zeros((_TAB_PAD - (_VOCAB + 1), 1), jnp.float32)])
    return _sc_call(inputs.T, tab.reshape(-1))[:, None]
